# baseline (speedup 1.0000x reference)
import functools

import jax
import jax.numpy as jnp
from jax import lax
from jax.experimental import pallas as pl
from jax.experimental.pallas import tpu as pltpu
from jax.experimental.pallas import tpu_sc as plsc

F32 = jnp.float32
N = 10000
NPAD = 10240
E = 160000
EPAD = 163840
D = 512
NC, NS, L = 2, 16, 16
RB = 1280
NBLK = NPAD // RB

_SC_MESH = plsc.VectorSubcoreMesh(
    core_axis_name="c", subcore_axis_name="s", num_cores=NC, num_subcores=NS)


def _mlp_body(x_ref, w1_ref, b1_ref, w2_ref, b2_ref, wg_ref, bg_ref, out_ref):
    h = jnp.maximum(
        jnp.dot(x_ref[...], w1_ref[...], preferred_element_type=F32)
        + b1_ref[...], 0.0)
    na = jnp.dot(h, w2_ref[...], preferred_element_type=F32) + b2_ref[...]
    xw = jnp.dot(na, wg_ref[...], preferred_element_type=F32) + bg_ref[...]
    for c in range(4):
        out_ref[c] = xw[:, c * 128:(c + 1) * 128]


def _mlp_call(x_pad, w1, b1, w2, b2, wg, bg):
    wspec = pl.BlockSpec((D, D), lambda i: (0, 0))
    bspec = pl.BlockSpec((1, D), lambda i: (0, 0))
    return pl.pallas_call(
        _mlp_body,
        grid=(NBLK,),
        in_specs=[pl.BlockSpec((RB, D), lambda i: (i, 0)),
                  wspec, bspec, wspec, bspec, wspec, bspec],
        out_specs=pl.BlockSpec((4, RB, 128), lambda i: (0, i, 0)),
        out_shape=jax.ShapeDtypeStruct((4, NPAD, 128), F32),
    )(x_pad, w1, b1, w2, b2, wg, bg)


_DEG_CB = 1600


@functools.partial(
    pl.kernel,
    out_type=jax.ShapeDtypeStruct((2 * NPAD,), F32),
    mesh=_SC_MESH,
    scratch_types=[
        pltpu.VMEM_SHARED((NPAD,), F32),
        pltpu.VMEM((_DEG_CB,), jnp.int32),
        pltpu.VMEM((_DEG_CB,), F32),
        pltpu.VMEM((320,), F32),
    ],
)
def _deg_kernel(dst_hbm, deg_hbm, hist_sh, didx_v, ones_v, slice_v):
    cid = lax.axis_index("c")
    sid = lax.axis_index("s")
    r0 = sid * 320
    for i in range(20):
        slice_v[pl.ds(i * 16, 16)] = jnp.zeros((16,), F32)
    pltpu.sync_copy(slice_v, hist_sh.at[pl.ds(r0, 320)])
    for i in range(_DEG_CB // 16):
        ones_v[pl.ds(i * 16, 16)] = jnp.full((16,), 1.0, F32)
    plsc.subcore_barrier()

    w = cid * NS + sid
    nb = jnp.where(w < 4, 4, 3)
    ebase = _DEG_CB * jnp.where(w < 4, w * 4, 16 + (w - 4) * 3)

    def chunk_body(k, _):
        pltpu.sync_copy(dst_hbm.at[pl.ds(ebase + k * _DEG_CB, _DEG_CB)],
                        didx_v)
        pltpu.sync_copy(ones_v, hist_sh.at[didx_v], add=True)
        return 0

    lax.fori_loop(0, nb, chunk_body, 0)
    plsc.subcore_barrier()
    pltpu.sync_copy(hist_sh.at[pl.ds(r0, 320)], slice_v)
    pltpu.sync_copy(slice_v, deg_hbm.at[pl.ds(cid * NPAD + r0, 320)])


def _scale_body(xw_ref, deg_ref, xwn_ref, rinv_ref):
    r = lax.rsqrt(deg_ref[:, 0:1] + deg_ref[:, 1:2] + 1.0)
    xwn_ref[0] = xw_ref[0] * r
    rinv_ref[...] = r


def _scale_call(xw4, deg2):
    return pl.pallas_call(
        _scale_body,
        grid=(4, NBLK),
        in_specs=[pl.BlockSpec((1, RB, 128), lambda c, i: (c, i, 0)),
                  pl.BlockSpec((RB, 2), lambda c, i: (i, 0))],
        out_specs=[pl.BlockSpec((1, RB, 128), lambda c, i: (c, i, 0)),
                   pl.BlockSpec((RB, 1), lambda c, i: (i, 0))],
        out_shape=[jax.ShapeDtypeStruct((4, NPAD, 128), F32),
                   jax.ShapeDtypeStruct((NPAD, 1), F32)],
    )(xw4, deg2)


_MP_NB = 80


@functools.partial(
    pl.kernel,
    out_type=jax.ShapeDtypeStruct((4 * NPAD, 128), F32),
    mesh=_SC_MESH,
    scratch_types=[
        pltpu.VMEM_SHARED((NPAD, 128), F32),
        pltpu.VMEM((128, 128), F32),
        pltpu.VMEM((128, 128), F32),
        pltpu.VMEM((64, 128), F32),
        pltpu.VMEM((128,), jnp.int32),
        pltpu.VMEM((128,), jnp.int32),
        pltpu.VMEM((128,), jnp.int32),
        pltpu.VMEM((128,), jnp.int32),
        pltpu.SemaphoreType.DMA,
        pltpu.SemaphoreType.DMA,
    ],
)
def _msgpass_kernel(xwn_hbm, srcoff_hbm, dst_hbm, emb_hbm,
                    s_sh, b0, b1, z_v, si0, si1, di0, di1,
                    sem0, sem1):
    cid = lax.axis_index("c")
    sid = lax.axis_index("s")
    bufs = (b0, b1)
    sis = (si0, si1)
    dis = (di0, di1)
    sems = (sem0, sem1)
    ebase = sid * (_MP_NB * 128)
    r0 = sid * 640
    for p in range(2):
        chunk = cid * 2 + p
        row_off = chunk * NPAD
        cbase = chunk * EPAD
        for i in range(64):
            for j in range(8):
                z_v[i, pl.ds(j * 16, 16)] = jnp.zeros((16,), F32)
        for k in range(10):
            pltpu.sync_copy(z_v, s_sh.at[pl.ds(r0 + k * 64, 64)])
        plsc.subcore_barrier()

        pltpu.sync_copy(srcoff_hbm.at[pl.ds(cbase + ebase, 128)], si0)
        pltpu.sync_copy(dst_hbm.at[pl.ds(ebase, 128)], di0)
        pltpu.async_copy(xwn_hbm.at[si0], b0, sem0)

        def pair_body(i, _):
            for par in range(2):
                b = i * 2 + par
                nxt = b + 1
                npar = 1 - par

                def prefetch():
                    e1 = ebase + nxt * 128
                    pltpu.sync_copy(srcoff_hbm.at[pl.ds(cbase + e1, 128)],
                                    sis[npar])
                    pltpu.sync_copy(dst_hbm.at[pl.ds(e1, 128)], dis[npar])
                    pltpu.async_copy(xwn_hbm.at[sis[npar]], bufs[npar],
                                     sems[npar])

                if par == 0:
                    prefetch()
                else:
                    pl.when(i < (_MP_NB // 2 - 1))(prefetch)
                pltpu.make_async_copy(xwn_hbm.at[sis[par]], bufs[par],
                                      sems[par]).wait()
                pltpu.sync_copy(bufs[par], s_sh.at[dis[par]], add=True)
            return 0

        lax.fori_loop(0, _MP_NB // 2, pair_body, 0)
        plsc.subcore_barrier()
        for k in range(10):
            rr = r0 + k * 64
            pltpu.sync_copy(s_sh.at[pl.ds(rr, 64)], z_v)
            pltpu.sync_copy(z_v, emb_hbm.at[pl.ds(row_off + rr, 64)])


def _heads_body(emb_ref, xwn_ref, rinv_ref, wn_ref, bn_ref, wsd_ref, bsd_ref,
                natt_ref, nia_ref, pn_ref, pc_ref, loss_ref):
    i = pl.program_id(0)
    c = pl.program_id(1)

    @pl.when(c == 0)
    def _():
        pn_ref[...] = jnp.broadcast_to(bn_ref[...], (RB, 128))
        pc_ref[...] = jnp.broadcast_to(bsd_ref[...], (RB, 128))

    @pl.when((i == 0) & (c == 0))
    def _():
        loss_ref[...] = jnp.zeros((1, 1), F32)

    e = jnp.maximum((emb_ref[0] + xwn_ref[0]) * rinv_ref[...], 0.0)
    pn_ref[...] += jnp.dot(e, wn_ref[...], preferred_element_type=F32)
    pc_ref[...] += jnp.dot(e, wsd_ref[...], preferred_element_type=F32)

    @pl.when(c == 3)
    def _():
        rows = i * RB + lax.broadcasted_iota(jnp.int32, (RB, 1), 0)
        valid = rows < N

        def bce_sum(z, t):
            v = (jnp.maximum(z, 0.0) - z * t
                 + jnp.log1p(jnp.exp(-jnp.abs(z))))
            return jnp.sum(jnp.where(valid, v, 0.0))

        s_att = bce_sum(pn_ref[:, 0:4], natt_ref[...])
        s_ia = bce_sum(pn_ref[:, 4:13], nia_ref[...])
        loss_ref[...] += s_att / (N * 4) + s_ia / (N * 9)


def _heads_call(emb4, xwn4, rinv2, wn, bn, wsd, bsd, natt_pad, nia_pad):
    return pl.pallas_call(
        _heads_body,
        grid=(NBLK, 4),
        in_specs=[pl.BlockSpec((1, RB, 128), lambda i, c: (c, i, 0)),
                  pl.BlockSpec((1, RB, 128), lambda i, c: (c, i, 0)),
                  pl.BlockSpec((RB, 1), lambda i, c: (i, 0)),
                  pl.BlockSpec((128, 128), lambda i, c: (c, 0)),
                  pl.BlockSpec((1, 128), lambda i, c: (0, 0)),
                  pl.BlockSpec((128, 128), lambda i, c: (c, 0)),
                  pl.BlockSpec((1, 128), lambda i, c: (0, 0)),
                  pl.BlockSpec((RB, 4), lambda i, c: (i, 0)),
                  pl.BlockSpec((RB, 9), lambda i, c: (i, 0))],
        out_specs=[pl.BlockSpec((RB, 128), lambda i, c: (i, 0)),
                   pl.BlockSpec((RB, 128), lambda i, c: (i, 0)),
                   pl.BlockSpec((1, 1), lambda i, c: (0, 0))],
        out_shape=[jax.ShapeDtypeStruct((NPAD, 128), F32),
                   jax.ShapeDtypeStruct((NPAD, 128), F32),
                   jax.ShapeDtypeStruct((1, 1), F32)],
    )(emb4, xwn4, rinv2, wn, bn, wsd, bsd, natt_pad, nia_pad)


_EH_NB = 40


@functools.partial(
    pl.kernel,
    out_type=jax.ShapeDtypeStruct((EPAD, 64), F32),
    mesh=_SC_MESH,
    scratch_types=[
        pltpu.VMEM((128,), jnp.int32), pltpu.VMEM((128,), jnp.int32),
        pltpu.VMEM((128,), jnp.int32), pltpu.VMEM((128,), jnp.int32),
        pltpu.VMEM((128, 128), F32), pltpu.VMEM((128, 128), F32),
        pltpu.VMEM((128, 128), F32), pltpu.VMEM((128, 128), F32),
        pltpu.VMEM((128, 64), F32),
        pltpu.SemaphoreType.DMA, pltpu.SemaphoreType.DMA,
        pltpu.SemaphoreType.DMA, pltpu.SemaphoreType.DMA,
    ],
)
def _edgehead_kernel(pc_hbm, src_hbm, dst_hbm, out_hbm,
                     si0, si1, di0, di1, gs0, gs1, gd0, gd1, ob_v,
                     sems0, sems1, semd0, semd1):
    cid = lax.axis_index("c")
    sid = lax.axis_index("s")
    sis = (si0, si1)
    dis = (di0, di1)
    gss = (gs0, gs1)
    gds = (gd0, gd1)
    semss = (sems0, sems1)
    semds = (semd0, semd1)
    w = cid * NS + sid
    ebase = w * (_EH_NB * 128)

    pltpu.sync_copy(src_hbm.at[pl.ds(ebase, 128)], si0)
    pltpu.sync_copy(dst_hbm.at[pl.ds(ebase, 128)], di0)
    pltpu.async_copy(pc_hbm.at[si0], gs0, sems0)
    pltpu.async_copy(pc_hbm.at[di0], gd0, semd0)

    def pair_body(i, _):
        for par in range(2):
            b = i * 2 + par
            nxt = b + 1
            npar = 1 - par

            def prefetch():
                e1 = ebase + nxt * 128
                pltpu.sync_copy(src_hbm.at[pl.ds(e1, 128)], sis[npar])
                pltpu.sync_copy(dst_hbm.at[pl.ds(e1, 128)], dis[npar])
                pltpu.async_copy(pc_hbm.at[sis[npar]], gss[npar],
                                 semss[npar])
                pltpu.async_copy(pc_hbm.at[dis[npar]], gds[npar],
                                 semds[npar])

            if par == 0:
                prefetch()
            else:
                pl.when(i < (_EH_NB // 2 - 1))(prefetch)
            pltpu.make_async_copy(pc_hbm.at[sis[par]], gss[par],
                                  semss[par]).wait()
            pltpu.make_async_copy(pc_hbm.at[dis[par]], gds[par],
                                  semds[par]).wait()
            g1, g2 = gss[par], gds[par]

            def row_body(r8, __):
                for r16 in range(8):
                    r = r8 * 8 + r16
                    for j in range(4):
                        ob_v[r, pl.ds(j * 16, 16)] = (
                            g1[r, pl.ds(j * 16, 16)]
                            + g2[r, pl.ds(64 + j * 16, 16)])
                return 0

            lax.fori_loop(0, 16, row_body, 0)
            pltpu.sync_copy(ob_v, out_hbm.at[pl.ds(ebase + b * 128, 128)])
        return 0

    lax.fori_loop(0, _EH_NB // 2, pair_body, 0)


_EL_RB = 2000


def _edgeloss_body(oe_ref, rel_ref, ta_ref, relp_ref, tap_ref, loss_ref):
    @pl.when(pl.program_id(0) == 0)
    def _():
        loss_ref[...] = jnp.zeros((1, 1), F32)

    def bce_sum(z, t):
        return jnp.sum(jnp.maximum(z, 0.0) - z * t
                       + jnp.log1p(jnp.exp(-jnp.abs(z))))

    zr = oe_ref[:, 0:19]
    zt = oe_ref[:, 19:52]
    relp_ref[...] = zr
    tap_ref[...] = zt
    loss_ref[...] += (bce_sum(zr, rel_ref[...]) / (E * 19)
                      + bce_sum(zt, ta_ref[...]) / (E * 33))


def _edgeloss_call(oe, erel, eta):
    return pl.pallas_call(
        _edgeloss_body,
        grid=(E // _EL_RB,),
        in_specs=[pl.BlockSpec((_EL_RB, 64), lambda i: (i, 0)),
                  pl.BlockSpec((_EL_RB, 19), lambda i: (i, 0)),
                  pl.BlockSpec((_EL_RB, 33), lambda i: (i, 0))],
        out_specs=[pl.BlockSpec((_EL_RB, 19), lambda i: (i, 0)),
                   pl.BlockSpec((_EL_RB, 33), lambda i: (i, 0)),
                   pl.BlockSpec((1, 1), lambda i: (0, 0))],
        out_shape=[jax.ShapeDtypeStruct((E, 19), F32),
                   jax.ShapeDtypeStruct((E, 33), F32),
                   jax.ShapeDtypeStruct((1, 1), F32)],
    )(oe, erel, eta)


def kernel(x, edge_index, node_att, node_ia, edge_rel, edge_ta,
           W_node1, b_node1, W_node2, b_node2, Wg, bg,
           W_att, b_att, W_ia, b_ia, W_rel, b_rel, W_ta, b_ta):
    src = edge_index[0]
    dst = edge_index[1]
    epad = jnp.full((EPAD - E,), NPAD - 1, jnp.int32)
    src_pad = jnp.concatenate([src, epad])
    dst_pad = jnp.concatenate([dst, epad])
    srcoff = (src_pad[None, :]
              + (jnp.arange(4, dtype=jnp.int32) * NPAD)[:, None]).reshape(-1)
    x_pad = jnp.concatenate([x, jnp.zeros((NPAD - N, D), F32)], axis=0)

    xw4 = _mlp_call(x_pad, W_node1, b_node1.reshape(1, D), W_node2,
                    b_node2.reshape(1, D), Wg, bg.reshape(1, D))
    deg = _deg_kernel(dst)
    xwn4, rinv2 = _scale_call(xw4, deg.reshape(2, NPAD).T)
    emb2 = _msgpass_kernel(xwn4.reshape(4 * NPAD, 128), srcoff, dst_pad)

    wn = jnp.concatenate([W_att, W_ia, jnp.zeros((D, 115), F32)], axis=1)
    bn = jnp.concatenate([b_att, b_ia, jnp.zeros((115,), F32)]).reshape(1, 128)
    z12 = jnp.zeros((D, 12), F32)
    wsd = jnp.concatenate([W_rel[:D], W_ta[:D], z12,
                           W_rel[D:], W_ta[D:], z12], axis=1)
    bsd = jnp.concatenate([b_rel, b_ta,
                           jnp.zeros((76,), F32)]).reshape(1, 128)
    natt_pad = jnp.concatenate([node_att, jnp.zeros((NPAD - N, 4), F32)],
                               axis=0)
    nia_pad = jnp.concatenate([node_ia, jnp.zeros((NPAD - N, 9), F32)],
                              axis=0)
    pn, pc, loss_n = _heads_call(emb2.reshape(4, NPAD, 128), xwn4, rinv2,
                                 wn, bn, wsd, bsd, natt_pad, nia_pad)
    oe = _edgehead_kernel(pc, src_pad, dst_pad)
    rel_pred, ta_pred, loss_e = _edgeloss_call(oe, edge_rel, edge_ta)

    loss = loss_n[0, 0] + loss_e[0, 0]
    return (loss, pn[:N, 0:4], pn[:N, 4:13], rel_pred, ta_pred)


# speedup vs baseline: 1.0172x; 1.0172x over previous
import functools

import jax
import jax.numpy as jnp
from jax import lax
from jax.experimental import pallas as pl
from jax.experimental.pallas import tpu as pltpu
from jax.experimental.pallas import tpu_sc as plsc

F32 = jnp.float32
N = 10000
NPAD = 10240
E = 160000
EPAD = 163840
D = 512
NC, NS, L = 2, 16, 16
RB = 1280
NBLK = NPAD // RB

_SC_MESH = plsc.VectorSubcoreMesh(
    core_axis_name="c", subcore_axis_name="s", num_cores=NC, num_subcores=NS)


def _mlp_body(x_ref, w1_ref, b1_ref, w2_ref, b2_ref, wg_ref, bg_ref, out_ref):
    h = jnp.maximum(
        jnp.dot(x_ref[...], w1_ref[...], preferred_element_type=F32)
        + b1_ref[...], 0.0)
    na = jnp.dot(h, w2_ref[...], preferred_element_type=F32) + b2_ref[...]
    xw = jnp.dot(na, wg_ref[...], preferred_element_type=F32) + bg_ref[...]
    for c in range(4):
        out_ref[c] = xw[:, c * 128:(c + 1) * 128]


def _mlp_call(x_pad, w1, b1, w2, b2, wg, bg):
    wspec = pl.BlockSpec((D, D), lambda i: (0, 0))
    bspec = pl.BlockSpec((1, D), lambda i: (0, 0))
    return pl.pallas_call(
        _mlp_body,
        grid=(NBLK,),
        in_specs=[pl.BlockSpec((RB, D), lambda i: (i, 0)),
                  wspec, bspec, wspec, bspec, wspec, bspec],
        out_specs=pl.BlockSpec((4, RB, 128), lambda i: (0, i, 0)),
        out_shape=jax.ShapeDtypeStruct((4, NPAD, 128), F32),
    )(x_pad, w1, b1, w2, b2, wg, bg)


_DEG_CB = 1600


@functools.partial(
    pl.kernel,
    out_type=jax.ShapeDtypeStruct((2 * NPAD,), F32),
    mesh=_SC_MESH,
    scratch_types=[
        pltpu.VMEM_SHARED((NPAD,), F32),
        pltpu.VMEM((_DEG_CB,), jnp.int32),
        pltpu.VMEM((_DEG_CB,), F32),
        pltpu.VMEM((320,), F32),
    ],
)
def _deg_kernel(dst_hbm, deg_hbm, hist_sh, didx_v, ones_v, slice_v):
    cid = lax.axis_index("c")
    sid = lax.axis_index("s")
    r0 = sid * 320
    for i in range(20):
        slice_v[pl.ds(i * 16, 16)] = jnp.zeros((16,), F32)
    pltpu.sync_copy(slice_v, hist_sh.at[pl.ds(r0, 320)])
    for i in range(_DEG_CB // 16):
        ones_v[pl.ds(i * 16, 16)] = jnp.full((16,), 1.0, F32)
    plsc.subcore_barrier()

    w = cid * NS + sid
    nb = jnp.where(w < 4, 4, 3)
    ebase = _DEG_CB * jnp.where(w < 4, w * 4, 16 + (w - 4) * 3)

    def chunk_body(k, _):
        pltpu.sync_copy(dst_hbm.at[pl.ds(ebase + k * _DEG_CB, _DEG_CB)],
                        didx_v)
        pltpu.sync_copy(ones_v, hist_sh.at[didx_v], add=True)
        return 0

    lax.fori_loop(0, nb, chunk_body, 0)
    plsc.subcore_barrier()
    pltpu.sync_copy(hist_sh.at[pl.ds(r0, 320)], slice_v)
    pltpu.sync_copy(slice_v, deg_hbm.at[pl.ds(cid * NPAD + r0, 320)])


def _scale_body(xw_ref, deg_ref, xwn_ref, rinv_ref):
    r = lax.rsqrt(deg_ref[:, 0:1] + deg_ref[:, 1:2] + 1.0)
    xwn_ref[0] = xw_ref[0] * r
    rinv_ref[...] = r


def _scale_call(xw4, deg2):
    return pl.pallas_call(
        _scale_body,
        grid=(4, NBLK),
        in_specs=[pl.BlockSpec((1, RB, 128), lambda c, i: (c, i, 0)),
                  pl.BlockSpec((RB, 2), lambda c, i: (i, 0))],
        out_specs=[pl.BlockSpec((1, RB, 128), lambda c, i: (c, i, 0)),
                   pl.BlockSpec((RB, 1), lambda c, i: (i, 0))],
        out_shape=[jax.ShapeDtypeStruct((4, NPAD, 128), F32),
                   jax.ShapeDtypeStruct((NPAD, 1), F32)],
    )(xw4, deg2)


_MP_NB = 80
_MP_SB = 20


@functools.partial(
    pl.kernel,
    out_type=jax.ShapeDtypeStruct((4 * NPAD, 128), F32),
    mesh=_SC_MESH,
    scratch_types=[
        pltpu.VMEM_SHARED((NPAD, 128), F32),
        pltpu.VMEM((128, 128), F32),
        pltpu.VMEM((128, 128), F32),
        pltpu.VMEM((64, 128), F32),
        pltpu.VMEM((_MP_SB * 128,), jnp.int32),
        pltpu.VMEM((_MP_SB * 128,), jnp.int32),
        pltpu.SemaphoreType.DMA,
        pltpu.SemaphoreType.DMA,
    ],
)
def _msgpass_kernel(xwn_hbm, srcoff_hbm, dst_hbm, emb_hbm,
                    s_sh, b0, b1, z_v, sidx_v, didx_v,
                    sem0, sem1):
    cid = lax.axis_index("c")
    sid = lax.axis_index("s")
    bufs = (b0, b1)
    sems = (sem0, sem1)
    ebase = sid * (_MP_NB * 128)
    r0 = sid * 640
    for p in range(2):
        chunk = cid * 2 + p
        row_off = chunk * NPAD
        cbase = chunk * EPAD
        for i in range(64):
            for j in range(8):
                z_v[i, pl.ds(j * 16, 16)] = jnp.zeros((16,), F32)
        for k in range(10):
            pltpu.sync_copy(z_v, s_sh.at[pl.ds(r0 + k * 64, 64)])
        plsc.subcore_barrier()

        def sb_body(sb, _):
            sbase = ebase + sb * (_MP_SB * 128)
            pltpu.sync_copy(
                srcoff_hbm.at[pl.ds(cbase + sbase, _MP_SB * 128)], sidx_v)
            pltpu.sync_copy(
                dst_hbm.at[pl.ds(sbase, _MP_SB * 128)], didx_v)
            pltpu.async_copy(xwn_hbm.at[sidx_v.at[pl.ds(0, 128)]], b0, sem0)

            def pair_body(i, _):
                for par in range(2):
                    b = i * 2 + par
                    nxt = b + 1
                    npar = 1 - par

                    def prefetch():
                        pltpu.async_copy(
                            xwn_hbm.at[sidx_v.at[pl.ds(nxt * 128, 128)]],
                            bufs[npar], sems[npar])

                    if par == 0:
                        prefetch()
                    else:
                        pl.when(i < (_MP_SB // 2 - 1))(prefetch)
                    pltpu.make_async_copy(
                        xwn_hbm.at[sidx_v.at[pl.ds(b * 128, 128)]],
                        bufs[par], sems[par]).wait()
                    pltpu.sync_copy(bufs[par],
                                    s_sh.at[didx_v.at[pl.ds(b * 128, 128)]],
                                    add=True)
                return 0

            lax.fori_loop(0, _MP_SB // 2, pair_body, 0)
            return 0

        lax.fori_loop(0, _MP_NB // _MP_SB, sb_body, 0)
        plsc.subcore_barrier()
        for k in range(10):
            rr = r0 + k * 64
            pltpu.sync_copy(s_sh.at[pl.ds(rr, 64)], z_v)
            pltpu.sync_copy(z_v, emb_hbm.at[pl.ds(row_off + rr, 64)])


def _heads_body(emb_ref, xwn_ref, rinv_ref, wn_ref, bn_ref, wsd_ref, bsd_ref,
                natt_ref, nia_ref, pn_ref, pc_ref, loss_ref):
    i = pl.program_id(0)
    c = pl.program_id(1)

    @pl.when(c == 0)
    def _():
        pn_ref[...] = jnp.broadcast_to(bn_ref[...], (RB, 128))
        pc_ref[...] = jnp.broadcast_to(bsd_ref[...], (RB, 128))

    @pl.when((i == 0) & (c == 0))
    def _():
        loss_ref[...] = jnp.zeros((1, 1), F32)

    e = jnp.maximum((emb_ref[0] + xwn_ref[0]) * rinv_ref[...], 0.0)
    pn_ref[...] += jnp.dot(e, wn_ref[...], preferred_element_type=F32)
    pc_ref[...] += jnp.dot(e, wsd_ref[...], preferred_element_type=F32)

    @pl.when(c == 3)
    def _():
        rows = i * RB + lax.broadcasted_iota(jnp.int32, (RB, 1), 0)
        valid = rows < N

        def bce_sum(z, t):
            v = (jnp.maximum(z, 0.0) - z * t
                 + jnp.log1p(jnp.exp(-jnp.abs(z))))
            return jnp.sum(jnp.where(valid, v, 0.0))

        s_att = bce_sum(pn_ref[:, 0:4], natt_ref[...])
        s_ia = bce_sum(pn_ref[:, 4:13], nia_ref[...])
        loss_ref[...] += s_att / (N * 4) + s_ia / (N * 9)


def _heads_call(emb4, xwn4, rinv2, wn, bn, wsd, bsd, natt_pad, nia_pad):
    return pl.pallas_call(
        _heads_body,
        grid=(NBLK, 4),
        in_specs=[pl.BlockSpec((1, RB, 128), lambda i, c: (c, i, 0)),
                  pl.BlockSpec((1, RB, 128), lambda i, c: (c, i, 0)),
                  pl.BlockSpec((RB, 1), lambda i, c: (i, 0)),
                  pl.BlockSpec((128, 128), lambda i, c: (c, 0)),
                  pl.BlockSpec((1, 128), lambda i, c: (0, 0)),
                  pl.BlockSpec((128, 128), lambda i, c: (c, 0)),
                  pl.BlockSpec((1, 128), lambda i, c: (0, 0)),
                  pl.BlockSpec((RB, 4), lambda i, c: (i, 0)),
                  pl.BlockSpec((RB, 9), lambda i, c: (i, 0))],
        out_specs=[pl.BlockSpec((RB, 128), lambda i, c: (i, 0)),
                   pl.BlockSpec((RB, 128), lambda i, c: (i, 0)),
                   pl.BlockSpec((1, 1), lambda i, c: (0, 0))],
        out_shape=[jax.ShapeDtypeStruct((NPAD, 128), F32),
                   jax.ShapeDtypeStruct((NPAD, 128), F32),
                   jax.ShapeDtypeStruct((1, 1), F32)],
    )(emb4, xwn4, rinv2, wn, bn, wsd, bsd, natt_pad, nia_pad)


_EH_NB = 40


@functools.partial(
    pl.kernel,
    out_type=jax.ShapeDtypeStruct((EPAD, 64), F32),
    mesh=_SC_MESH,
    scratch_types=[
        pltpu.VMEM((_EH_NB * 128,), jnp.int32),
        pltpu.VMEM((_EH_NB * 128,), jnp.int32),
        pltpu.VMEM((128, 128), F32), pltpu.VMEM((128, 128), F32),
        pltpu.VMEM((128, 128), F32), pltpu.VMEM((128, 128), F32),
        pltpu.VMEM((128, 64), F32),
        pltpu.SemaphoreType.DMA, pltpu.SemaphoreType.DMA,
        pltpu.SemaphoreType.DMA, pltpu.SemaphoreType.DMA,
    ],
)
def _edgehead_kernel(pc_hbm, src_hbm, dst_hbm, out_hbm,
                     sidx_v, didx_v, gs0, gs1, gd0, gd1, ob_v,
                     sems0, sems1, semd0, semd1):
    cid = lax.axis_index("c")
    sid = lax.axis_index("s")
    gss = (gs0, gs1)
    gds = (gd0, gd1)
    semss = (sems0, sems1)
    semds = (semd0, semd1)
    w = cid * NS + sid
    ebase = w * (_EH_NB * 128)

    pltpu.sync_copy(src_hbm.at[pl.ds(ebase, _EH_NB * 128)], sidx_v)
    pltpu.sync_copy(dst_hbm.at[pl.ds(ebase, _EH_NB * 128)], didx_v)
    pltpu.async_copy(pc_hbm.at[sidx_v.at[pl.ds(0, 128)]], gs0, sems0)
    pltpu.async_copy(pc_hbm.at[didx_v.at[pl.ds(0, 128)]], gd0, semd0)

    def pair_body(i, _):
        for par in range(2):
            b = i * 2 + par
            nxt = b + 1
            npar = 1 - par

            def prefetch():
                pltpu.async_copy(pc_hbm.at[sidx_v.at[pl.ds(nxt * 128, 128)]],
                                 gss[npar], semss[npar])
                pltpu.async_copy(pc_hbm.at[didx_v.at[pl.ds(nxt * 128, 128)]],
                                 gds[npar], semds[npar])

            if par == 0:
                prefetch()
            else:
                pl.when(i < (_EH_NB // 2 - 1))(prefetch)
            pltpu.make_async_copy(pc_hbm.at[sidx_v.at[pl.ds(b * 128, 128)]],
                                  gss[par], semss[par]).wait()
            pltpu.make_async_copy(pc_hbm.at[didx_v.at[pl.ds(b * 128, 128)]],
                                  gds[par], semds[par]).wait()
            g1, g2 = gss[par], gds[par]

            def row_body(r8, __):
                for r16 in range(8):
                    r = r8 * 8 + r16
                    for j in range(4):
                        ob_v[r, pl.ds(j * 16, 16)] = (
                            g1[r, pl.ds(j * 16, 16)]
                            + g2[r, pl.ds(64 + j * 16, 16)])
                return 0

            lax.fori_loop(0, 16, row_body, 0)
            pltpu.sync_copy(ob_v, out_hbm.at[pl.ds(ebase + b * 128, 128)])
        return 0

    lax.fori_loop(0, _EH_NB // 2, pair_body, 0)


_EL_RB = 2000


def _edgeloss_body(oe_ref, rel_ref, ta_ref, relp_ref, tap_ref, loss_ref):
    @pl.when(pl.program_id(0) == 0)
    def _():
        loss_ref[...] = jnp.zeros((1, 1), F32)

    def bce_sum(z, t):
        return jnp.sum(jnp.maximum(z, 0.0) - z * t
                       + jnp.log1p(jnp.exp(-jnp.abs(z))))

    zr = oe_ref[:, 0:19]
    zt = oe_ref[:, 19:52]
    relp_ref[...] = zr
    tap_ref[...] = zt
    loss_ref[...] += (bce_sum(zr, rel_ref[...]) / (E * 19)
                      + bce_sum(zt, ta_ref[...]) / (E * 33))


def _edgeloss_call(oe, erel, eta):
    return pl.pallas_call(
        _edgeloss_body,
        grid=(E // _EL_RB,),
        in_specs=[pl.BlockSpec((_EL_RB, 64), lambda i: (i, 0)),
                  pl.BlockSpec((_EL_RB, 19), lambda i: (i, 0)),
                  pl.BlockSpec((_EL_RB, 33), lambda i: (i, 0))],
        out_specs=[pl.BlockSpec((_EL_RB, 19), lambda i: (i, 0)),
                   pl.BlockSpec((_EL_RB, 33), lambda i: (i, 0)),
                   pl.BlockSpec((1, 1), lambda i: (0, 0))],
        out_shape=[jax.ShapeDtypeStruct((E, 19), F32),
                   jax.ShapeDtypeStruct((E, 33), F32),
                   jax.ShapeDtypeStruct((1, 1), F32)],
    )(oe, erel, eta)


def kernel(x, edge_index, node_att, node_ia, edge_rel, edge_ta,
           W_node1, b_node1, W_node2, b_node2, Wg, bg,
           W_att, b_att, W_ia, b_ia, W_rel, b_rel, W_ta, b_ta):
    src = edge_index[0]
    dst = edge_index[1]
    epad = jnp.full((EPAD - E,), NPAD - 1, jnp.int32)
    src_pad = jnp.concatenate([src, epad])
    dst_pad = jnp.concatenate([dst, epad])
    srcoff = (src_pad[None, :]
              + (jnp.arange(4, dtype=jnp.int32) * NPAD)[:, None]).reshape(-1)
    x_pad = jnp.concatenate([x, jnp.zeros((NPAD - N, D), F32)], axis=0)

    xw4 = _mlp_call(x_pad, W_node1, b_node1.reshape(1, D), W_node2,
                    b_node2.reshape(1, D), Wg, bg.reshape(1, D))
    deg = _deg_kernel(dst)
    xwn4, rinv2 = _scale_call(xw4, deg.reshape(2, NPAD).T)
    emb2 = _msgpass_kernel(xwn4.reshape(4 * NPAD, 128), srcoff, dst_pad)

    wn = jnp.concatenate([W_att, W_ia, jnp.zeros((D, 115), F32)], axis=1)
    bn = jnp.concatenate([b_att, b_ia, jnp.zeros((115,), F32)]).reshape(1, 128)
    z12 = jnp.zeros((D, 12), F32)
    wsd = jnp.concatenate([W_rel[:D], W_ta[:D], z12,
                           W_rel[D:], W_ta[D:], z12], axis=1)
    bsd = jnp.concatenate([b_rel, b_ta,
                           jnp.zeros((76,), F32)]).reshape(1, 128)
    natt_pad = jnp.concatenate([node_att, jnp.zeros((NPAD - N, 4), F32)],
                               axis=0)
    nia_pad = jnp.concatenate([node_ia, jnp.zeros((NPAD - N, 9), F32)],
                              axis=0)
    pn, pc, loss_n = _heads_call(emb2.reshape(4, NPAD, 128), xwn4, rinv2,
                                 wn, bn, wsd, bsd, natt_pad, nia_pad)
    oe = _edgehead_kernel(pc, src_pad, dst_pad)
    rel_pred, ta_pred, loss_e = _edgeloss_call(oe, edge_rel, edge_ta)

    loss = loss_n[0, 0] + loss_e[0, 0]
    return (loss, pn[:N, 0:4], pn[:N, 4:13], rel_pred, ta_pred)


# speedup vs baseline: 1.3753x; 1.3520x over previous
import functools

import jax
import jax.numpy as jnp
from jax import lax
from jax.experimental import pallas as pl
from jax.experimental.pallas import tpu as pltpu
from jax.experimental.pallas import tpu_sc as plsc

F32 = jnp.float32
N = 10000
NPAD = 10240
E = 160000
D = 512
NC, NS, L = 2, 16, 16
RB = 1280
NBLK = NPAD // RB

_SC_MESH = plsc.VectorSubcoreMesh(
    core_axis_name="c", subcore_axis_name="s", num_cores=NC, num_subcores=NS)


def _mlp_body(x_ref, w1_ref, b1_ref, w2_ref, b2_ref, wg_ref, bg_ref, out_ref):
    h = jnp.maximum(
        jnp.dot(x_ref[...], w1_ref[...], preferred_element_type=F32)
        + b1_ref[...], 0.0)
    na = jnp.dot(h, w2_ref[...], preferred_element_type=F32) + b2_ref[...]
    xw = jnp.dot(na, wg_ref[...], preferred_element_type=F32) + bg_ref[...]
    for c in range(4):
        out_ref[c] = xw[:, c * 128:(c + 1) * 128]


def _mlp_call(x, w1, b1, w2, b2, wg, bg):
    wspec = pl.BlockSpec((D, D), lambda i: (0, 0))
    bspec = pl.BlockSpec((1, D), lambda i: (0, 0))
    return pl.pallas_call(
        _mlp_body,
        grid=(25,),
        in_specs=[pl.BlockSpec((400, D), lambda i: (i, 0)),
                  wspec, bspec, wspec, bspec, wspec, bspec],
        out_specs=pl.BlockSpec((4, 400, 128), lambda i: (0, i, 0)),
        out_shape=jax.ShapeDtypeStruct((4, NPAD, 128), F32),
    )(x, w1, b1, w2, b2, wg, bg)


_DEG_CB = 1600


@functools.partial(
    pl.kernel,
    out_type=jax.ShapeDtypeStruct((2 * NPAD,), F32),
    mesh=_SC_MESH,
    scratch_types=[
        pltpu.VMEM_SHARED((NPAD,), F32),
        pltpu.VMEM((_DEG_CB,), jnp.int32),
        pltpu.VMEM((_DEG_CB,), F32),
        pltpu.VMEM((320,), F32),
    ],
)
def _deg_kernel(dst_hbm, deg_hbm, hist_sh, didx_v, ones_v, slice_v):
    cid = lax.axis_index("c")
    sid = lax.axis_index("s")
    r0 = sid * 320
    for i in range(20):
        slice_v[pl.ds(i * 16, 16)] = jnp.zeros((16,), F32)
    pltpu.sync_copy(slice_v, hist_sh.at[pl.ds(r0, 320)])
    for i in range(_DEG_CB // 16):
        ones_v[pl.ds(i * 16, 16)] = jnp.full((16,), 1.0, F32)
    plsc.subcore_barrier()

    w = cid * NS + sid
    nb = jnp.where(w < 4, 4, 3)
    ebase = _DEG_CB * jnp.where(w < 4, w * 4, 16 + (w - 4) * 3)

    def chunk_body(k, _):
        pltpu.sync_copy(dst_hbm.at[pl.ds(ebase + k * _DEG_CB, _DEG_CB)],
                        didx_v)
        pltpu.sync_copy(ones_v, hist_sh.at[didx_v], add=True)
        return 0

    lax.fori_loop(0, nb, chunk_body, 0)
    plsc.subcore_barrier()
    pltpu.sync_copy(hist_sh.at[pl.ds(r0, 320)], slice_v)
    pltpu.sync_copy(slice_v, deg_hbm.at[pl.ds(cid * NPAD + r0, 320)])


def _scale_body(xw_ref, deg_ref, xwn_ref, rinv_ref):
    r = lax.rsqrt(deg_ref[:, 0:1] + deg_ref[:, 1:2] + 1.0)
    xwn_ref[0] = xw_ref[0] * r
    rinv_ref[...] = r


def _scale_call(xw4, deg2):
    return pl.pallas_call(
        _scale_body,
        grid=(4, NBLK),
        in_specs=[pl.BlockSpec((1, RB, 128), lambda c, i: (c, i, 0)),
                  pl.BlockSpec((RB, 2), lambda c, i: (i, 0))],
        out_specs=[pl.BlockSpec((1, RB, 128), lambda c, i: (c, i, 0)),
                   pl.BlockSpec((RB, 1), lambda c, i: (i, 0))],
        out_shape=[jax.ShapeDtypeStruct((4, NPAD, 128), F32),
                   jax.ShapeDtypeStruct((NPAD, 1), F32)],
    )(xw4, deg2)


@functools.partial(
    pl.kernel,
    out_type=jax.ShapeDtypeStruct((4 * NPAD, 128), F32),
    mesh=_SC_MESH,
    scratch_types=[
        pltpu.VMEM_SHARED((NPAD, 128), F32),
        pltpu.VMEM((128, 128), F32),
        pltpu.VMEM((160, 128), F32),
        pltpu.VMEM((128,), jnp.int32),
        pltpu.VMEM((128,), jnp.int32),
        pltpu.SemaphoreType.DMA,
    ],
)
def _msgpass_kernel(xwn_hbm, src_hbm, dst_hbm, emb_hbm,
                    s_sh, buf_v, fin_v, sidx_v, didx_v, sem):
    cid = lax.axis_index("c")
    sid = lax.axis_index("s")
    nb = jnp.where(sid < 2, 79, 78)
    ebase = 128 * jnp.where(sid < 2, sid * 79, 158 + (sid - 2) * 78)
    r0 = sid * 640
    for p in range(2):
        row_off = (cid * 2 + p) * NPAD
        for k in range(4):
            rr = r0 + k * 160
            pltpu.sync_copy(xwn_hbm.at[pl.ds(row_off + rr, 160)], fin_v)
            pltpu.sync_copy(fin_v, s_sh.at[pl.ds(rr, 160)])
        plsc.subcore_barrier()

        def batch_body(b, _):
            e0 = ebase + b * 128
            pltpu.sync_copy(src_hbm.at[pl.ds(e0, 128)], sidx_v)
            pltpu.sync_copy(dst_hbm.at[pl.ds(e0, 128)], didx_v)
            for i in range(8):
                sidx_v[pl.ds(i * 16, 16)] = sidx_v[pl.ds(i * 16, 16)] + row_off
            pltpu.async_copy(xwn_hbm.at[sidx_v], buf_v, sem).wait()
            pltpu.sync_copy(buf_v, s_sh.at[didx_v], add=True)
            return 0

        lax.fori_loop(0, nb, batch_body, 0)
        plsc.subcore_barrier()
        for k in range(4):
            rr = r0 + k * 160
            pltpu.sync_copy(s_sh.at[pl.ds(rr, 160)], fin_v)
            pltpu.sync_copy(fin_v, emb_hbm.at[pl.ds(row_off + rr, 160)])
        plsc.subcore_barrier()


def _heads_body(emb_ref, rinv_ref, wn_ref, bn_ref, wsd_ref, bsd_ref,
                natt_ref, nia_ref, pn_ref, pc_ref, loss_ref):
    i = pl.program_id(0)
    c = pl.program_id(1)

    @pl.when(c == 0)
    def _():
        pn_ref[...] = jnp.broadcast_to(bn_ref[...], (RB, 128))
        pc_ref[...] = jnp.broadcast_to(bsd_ref[...], (RB, 128))

    @pl.when((i == 0) & (c == 0))
    def _():
        loss_ref[...] = jnp.zeros((1, 1), F32)

    e = jnp.maximum(emb_ref[0] * rinv_ref[...], 0.0)
    pn_ref[...] += jnp.dot(e, wn_ref[...], preferred_element_type=F32)
    pc_ref[...] += jnp.dot(e, wsd_ref[...], preferred_element_type=F32)

    @pl.when(c == 3)
    def _():
        rows = i * RB + lax.broadcasted_iota(jnp.int32, (RB, 1), 0)
        valid = rows < N

        def bce_sum(z, t):
            v = (jnp.maximum(z, 0.0) - z * t
                 + jnp.log1p(jnp.exp(-jnp.abs(z))))
            return jnp.sum(jnp.where(valid, v, 0.0))

        s_att = bce_sum(pn_ref[:, 0:4], natt_ref[...])
        s_ia = bce_sum(pn_ref[:, 4:13], nia_ref[...])
        loss_ref[...] += s_att / (N * 4) + s_ia / (N * 9)


def _heads_call(emb4, rinv2, wn, bn, wsd, bsd, natt_pad, nia_pad):
    return pl.pallas_call(
        _heads_body,
        grid=(NBLK, 4),
        in_specs=[pl.BlockSpec((1, RB, 128), lambda i, c: (c, i, 0)),
                  pl.BlockSpec((RB, 1), lambda i, c: (i, 0)),
                  pl.BlockSpec((128, 128), lambda i, c: (c, 0)),
                  pl.BlockSpec((1, 128), lambda i, c: (0, 0)),
                  pl.BlockSpec((128, 128), lambda i, c: (c, 0)),
                  pl.BlockSpec((1, 128), lambda i, c: (0, 0)),
                  pl.BlockSpec((RB, 4), lambda i, c: (i, 0)),
                  pl.BlockSpec((RB, 9), lambda i, c: (i, 0))],
        out_specs=[pl.BlockSpec((RB, 128), lambda i, c: (i, 0)),
                   pl.BlockSpec((RB, 128), lambda i, c: (i, 0)),
                   pl.BlockSpec((1, 1), lambda i, c: (0, 0))],
        out_shape=[jax.ShapeDtypeStruct((NPAD, 128), F32),
                   jax.ShapeDtypeStruct((NPAD, 128), F32),
                   jax.ShapeDtypeStruct((1, 1), F32)],
    )(emb4, rinv2, wn, bn, wsd, bsd, natt_pad, nia_pad)


@functools.partial(
    pl.kernel,
    out_type=jax.ShapeDtypeStruct((E, 64), F32),
    mesh=_SC_MESH,
    scratch_types=[
        pltpu.VMEM((128,), jnp.int32), pltpu.VMEM((128,), jnp.int32),
        pltpu.VMEM((128, 128), F32), pltpu.VMEM((128, 128), F32),
        pltpu.VMEM((128, 64), F32),
        pltpu.SemaphoreType.DMA, pltpu.SemaphoreType.DMA,
    ],
)
def _edgehead_kernel(pc_hbm, src_hbm, dst_hbm, out_hbm,
                     sidx_v, didx_v, g1_v, g2_v, ob_v, sem1, sem2):
    cid = lax.axis_index("c")
    sid = lax.axis_index("s")
    w = cid * NS + sid
    nb = jnp.where(w < 2, 40, 39)
    ebase = 128 * jnp.where(w < 2, w * 40, 80 + (w - 2) * 39)

    def batch_body(b, _):
        e0 = ebase + b * 128
        pltpu.sync_copy(src_hbm.at[pl.ds(e0, 128)], sidx_v)
        pltpu.sync_copy(dst_hbm.at[pl.ds(e0, 128)], didx_v)
        cp1 = pltpu.async_copy(pc_hbm.at[sidx_v], g1_v, sem1)
        cp2 = pltpu.async_copy(pc_hbm.at[didx_v], g2_v, sem2)
        cp1.wait()
        cp2.wait()

        def row_body(r, __):
            for j in range(4):
                ob_v[r, pl.ds(j * 16, 16)] = (
                    g1_v[r, pl.ds(j * 16, 16)]
                    + g2_v[r, pl.ds(64 + j * 16, 16)])
            return 0

        lax.fori_loop(0, 128, row_body, 0)
        pltpu.sync_copy(ob_v, out_hbm.at[pl.ds(e0, 128)])
        return 0

    lax.fori_loop(0, nb, batch_body, 0)


_EL_RB = 2000


def _edgeloss_body(oe_ref, rel_ref, ta_ref, relp_ref, tap_ref, loss_ref):
    @pl.when(pl.program_id(0) == 0)
    def _():
        loss_ref[...] = jnp.zeros((1, 1), F32)

    def bce_sum(z, t):
        return jnp.sum(jnp.maximum(z, 0.0) - z * t
                       + jnp.log1p(jnp.exp(-jnp.abs(z))))

    zr = oe_ref[:, 0:19]
    zt = oe_ref[:, 19:52]
    relp_ref[...] = zr
    tap_ref[...] = zt
    loss_ref[...] += (bce_sum(zr, rel_ref[...]) / (E * 19)
                      + bce_sum(zt, ta_ref[...]) / (E * 33))


def _edgeloss_call(oe, erel, eta):
    return pl.pallas_call(
        _edgeloss_body,
        grid=(E // _EL_RB,),
        in_specs=[pl.BlockSpec((_EL_RB, 64), lambda i: (i, 0)),
                  pl.BlockSpec((_EL_RB, 19), lambda i: (i, 0)),
                  pl.BlockSpec((_EL_RB, 33), lambda i: (i, 0))],
        out_specs=[pl.BlockSpec((_EL_RB, 19), lambda i: (i, 0)),
                   pl.BlockSpec((_EL_RB, 33), lambda i: (i, 0)),
                   pl.BlockSpec((1, 1), lambda i: (0, 0))],
        out_shape=[jax.ShapeDtypeStruct((E, 19), F32),
                   jax.ShapeDtypeStruct((E, 33), F32),
                   jax.ShapeDtypeStruct((1, 1), F32)],
    )(oe, erel, eta)


def kernel(x, edge_index, node_att, node_ia, edge_rel, edge_ta,
           W_node1, b_node1, W_node2, b_node2, Wg, bg,
           W_att, b_att, W_ia, b_ia, W_rel, b_rel, W_ta, b_ta):
    src = edge_index[0]
    dst = edge_index[1]

    xw4 = _mlp_call(x, W_node1, b_node1.reshape(1, D), W_node2,
                    b_node2.reshape(1, D), Wg, bg.reshape(1, D))
    deg = _deg_kernel(dst)
    xwn4, rinv2 = _scale_call(xw4, deg.reshape(2, NPAD).T)
    emb2 = _msgpass_kernel(xwn4.reshape(4 * NPAD, 128), src, dst)

    wn = jnp.concatenate([W_att, W_ia, jnp.zeros((D, 115), F32)], axis=1)
    bn = jnp.concatenate([b_att, b_ia, jnp.zeros((115,), F32)]).reshape(1, 128)
    z12 = jnp.zeros((D, 12), F32)
    wsd = jnp.concatenate([W_rel[:D], W_ta[:D], z12,
                           W_rel[D:], W_ta[D:], z12], axis=1)
    bsd = jnp.concatenate([b_rel, b_ta,
                           jnp.zeros((76,), F32)]).reshape(1, 128)
    natt_pad = jnp.concatenate([node_att, jnp.zeros((NPAD - N, 4), F32)],
                               axis=0)
    nia_pad = jnp.concatenate([node_ia, jnp.zeros((NPAD - N, 9), F32)],
                              axis=0)
    pn, pc, loss_n = _heads_call(emb2.reshape(4, NPAD, 128), rinv2,
                                 wn, bn, wsd, bsd, natt_pad, nia_pad)
    oe = _edgehead_kernel(pc, src, dst)
    rel_pred, ta_pred, loss_e = _edgeloss_call(oe, edge_rel, edge_ta)

    loss = loss_n[0, 0] + loss_e[0, 0]
    return (loss, pn[:N, 0:4], pn[:N, 4:13], rel_pred, ta_pred)


# speedup vs baseline: 1.5240x; 1.1081x over previous
import functools

import jax
import jax.numpy as jnp
from jax import lax
from jax.experimental import pallas as pl
from jax.experimental.pallas import tpu as pltpu
from jax.experimental.pallas import tpu_sc as plsc

F32 = jnp.float32
N = 10000
NPAD = 10240
E = 160000
D = 512
NC, NS, L = 2, 16, 16
RB = 1280
NBLK = NPAD // RB

_SC_MESH = plsc.VectorSubcoreMesh(
    core_axis_name="c", subcore_axis_name="s", num_cores=NC, num_subcores=NS)


def _mlp_body(x_ref, w1_ref, b1_ref, w2_ref, b2_ref, wg_ref, bg_ref, out_ref):
    h = jnp.maximum(
        jnp.dot(x_ref[...], w1_ref[...], preferred_element_type=F32)
        + b1_ref[...], 0.0)
    na = jnp.dot(h, w2_ref[...], preferred_element_type=F32) + b2_ref[...]
    xw = jnp.dot(na, wg_ref[...], preferred_element_type=F32) + bg_ref[...]
    for c in range(4):
        out_ref[c] = xw[:, c * 128:(c + 1) * 128]


def _mlp_call(x, w1, b1, w2, b2, wg, bg):
    wspec = pl.BlockSpec((D, D), lambda i: (0, 0))
    bspec = pl.BlockSpec((1, D), lambda i: (0, 0))
    return pl.pallas_call(
        _mlp_body,
        grid=(25,),
        in_specs=[pl.BlockSpec((400, D), lambda i: (i, 0)),
                  wspec, bspec, wspec, bspec, wspec, bspec],
        out_specs=pl.BlockSpec((4, 400, 128), lambda i: (0, i, 0)),
        out_shape=jax.ShapeDtypeStruct((4, NPAD, 128), F32),
    )(x, w1, b1, w2, b2, wg, bg)


_DEG_CB = 1600


@functools.partial(
    pl.kernel,
    out_type=jax.ShapeDtypeStruct((2 * NPAD,), F32),
    mesh=_SC_MESH,
    scratch_types=[
        pltpu.VMEM_SHARED((NPAD,), F32),
        pltpu.VMEM((_DEG_CB,), jnp.int32),
        pltpu.VMEM((_DEG_CB,), F32),
        pltpu.VMEM((320,), F32),
    ],
)
def _deg_kernel(dst_hbm, deg_hbm, hist_sh, didx_v, ones_v, slice_v):
    cid = lax.axis_index("c")
    sid = lax.axis_index("s")
    r0 = sid * 320
    for i in range(20):
        slice_v[pl.ds(i * 16, 16)] = jnp.zeros((16,), F32)
    pltpu.sync_copy(slice_v, hist_sh.at[pl.ds(r0, 320)])
    for i in range(_DEG_CB // 16):
        ones_v[pl.ds(i * 16, 16)] = jnp.full((16,), 1.0, F32)
    plsc.subcore_barrier()

    w = cid * NS + sid
    nb = jnp.where(w < 4, 4, 3)
    ebase = _DEG_CB * jnp.where(w < 4, w * 4, 16 + (w - 4) * 3)

    def chunk_body(k, _):
        pltpu.sync_copy(dst_hbm.at[pl.ds(ebase + k * _DEG_CB, _DEG_CB)],
                        didx_v)
        pltpu.sync_copy(ones_v, hist_sh.at[didx_v], add=True)
        return 0

    lax.fori_loop(0, nb, chunk_body, 0)
    plsc.subcore_barrier()
    pltpu.sync_copy(hist_sh.at[pl.ds(r0, 320)], slice_v)
    pltpu.sync_copy(slice_v, deg_hbm.at[pl.ds(cid * NPAD + r0, 320)])


def _scale_body(xw_ref, deg_ref, xwn_ref, rinv_ref):
    r = lax.rsqrt(deg_ref[:, 0:1] + deg_ref[:, 1:2] + 1.0)
    xwn_ref[0] = xw_ref[0] * r
    rinv_ref[...] = r


def _scale_call(xw4, deg2):
    return pl.pallas_call(
        _scale_body,
        grid=(4, NBLK),
        in_specs=[pl.BlockSpec((1, RB, 128), lambda c, i: (c, i, 0)),
                  pl.BlockSpec((RB, 2), lambda c, i: (i, 0))],
        out_specs=[pl.BlockSpec((1, RB, 128), lambda c, i: (c, i, 0)),
                   pl.BlockSpec((RB, 1), lambda c, i: (i, 0))],
        out_shape=[jax.ShapeDtypeStruct((4, NPAD, 128), F32),
                   jax.ShapeDtypeStruct((NPAD, 1), F32)],
    )(xw4, deg2)


_MP_B = 256


@functools.partial(
    pl.kernel,
    out_type=jax.ShapeDtypeStruct((4 * NPAD, 128), F32),
    mesh=_SC_MESH,
    scratch_types=[
        pltpu.VMEM_SHARED((NPAD, 128), F32),
        pltpu.VMEM((_MP_B, 128), F32),
        pltpu.VMEM((64, 128), F32),
        pltpu.VMEM((_MP_B,), jnp.int32),
        pltpu.VMEM((_MP_B,), jnp.int32),
        pltpu.SemaphoreType.DMA,
    ],
)
def _msgpass_kernel(xwn_hbm, src_hbm, dst_hbm, emb_hbm,
                    s_sh, buf_v, fin_v, sidx_v, didx_v, sem):
    cid = lax.axis_index("c")
    sid = lax.axis_index("s")
    nb = jnp.where(sid < 1, 40, 39)
    ebase = _MP_B * jnp.where(sid < 1, 0, 40 + (sid - 1) * 39)
    r0 = sid * 640
    for p in range(2):
        row_off = (cid * 2 + p) * NPAD
        for k in range(10):
            rr = r0 + k * 64
            pltpu.sync_copy(xwn_hbm.at[pl.ds(row_off + rr, 64)], fin_v)
            pltpu.sync_copy(fin_v, s_sh.at[pl.ds(rr, 64)])
        plsc.subcore_barrier()

        def batch_body(b, _):
            e0 = ebase + b * _MP_B
            pltpu.sync_copy(src_hbm.at[pl.ds(e0, _MP_B)], sidx_v)
            pltpu.sync_copy(dst_hbm.at[pl.ds(e0, _MP_B)], didx_v)
            for i in range(_MP_B // 16):
                sidx_v[pl.ds(i * 16, 16)] = sidx_v[pl.ds(i * 16, 16)] + row_off
            pltpu.async_copy(xwn_hbm.at[sidx_v], buf_v, sem).wait()
            pltpu.sync_copy(buf_v, s_sh.at[didx_v], add=True)
            return 0

        lax.fori_loop(0, nb, batch_body, 0)
        plsc.subcore_barrier()
        for k in range(10):
            rr = r0 + k * 64
            pltpu.sync_copy(s_sh.at[pl.ds(rr, 64)], fin_v)
            pltpu.sync_copy(fin_v, emb_hbm.at[pl.ds(row_off + rr, 64)])
        plsc.subcore_barrier()


def _heads_body(emb_ref, rinv_ref, wn_ref, bn_ref, wsd_ref, bsd_ref,
                natt_ref, nia_ref, pn_ref, pc_ref, loss_ref):
    i = pl.program_id(0)
    c = pl.program_id(1)

    @pl.when(c == 0)
    def _():
        pn_ref[...] = jnp.broadcast_to(bn_ref[...], (RB, 128))
        pc_ref[...] = jnp.broadcast_to(bsd_ref[...], (RB, 128))

    @pl.when((i == 0) & (c == 0))
    def _():
        loss_ref[...] = jnp.zeros((1, 1), F32)

    e = jnp.maximum(emb_ref[0] * rinv_ref[...], 0.0)
    pn_ref[...] += jnp.dot(e, wn_ref[...], preferred_element_type=F32)
    pc_ref[...] += jnp.dot(e, wsd_ref[...], preferred_element_type=F32)

    @pl.when(c == 3)
    def _():
        rows = i * RB + lax.broadcasted_iota(jnp.int32, (RB, 1), 0)
        valid = rows < N

        def bce_sum(z, t):
            v = (jnp.maximum(z, 0.0) - z * t
                 + jnp.log1p(jnp.exp(-jnp.abs(z))))
            return jnp.sum(jnp.where(valid, v, 0.0))

        s_att = bce_sum(pn_ref[:, 0:4], natt_ref[...])
        s_ia = bce_sum(pn_ref[:, 4:13], nia_ref[...])
        loss_ref[...] += s_att / (N * 4) + s_ia / (N * 9)


def _heads_call(emb4, rinv2, wn, bn, wsd, bsd, natt_pad, nia_pad):
    return pl.pallas_call(
        _heads_body,
        grid=(NBLK, 4),
        in_specs=[pl.BlockSpec((1, RB, 128), lambda i, c: (c, i, 0)),
                  pl.BlockSpec((RB, 1), lambda i, c: (i, 0)),
                  pl.BlockSpec((128, 128), lambda i, c: (c, 0)),
                  pl.BlockSpec((1, 128), lambda i, c: (0, 0)),
                  pl.BlockSpec((128, 128), lambda i, c: (c, 0)),
                  pl.BlockSpec((1, 128), lambda i, c: (0, 0)),
                  pl.BlockSpec((RB, 4), lambda i, c: (i, 0)),
                  pl.BlockSpec((RB, 9), lambda i, c: (i, 0))],
        out_specs=[pl.BlockSpec((RB, 128), lambda i, c: (i, 0)),
                   pl.BlockSpec((RB, 128), lambda i, c: (i, 0)),
                   pl.BlockSpec((1, 1), lambda i, c: (0, 0))],
        out_shape=[jax.ShapeDtypeStruct((NPAD, 128), F32),
                   jax.ShapeDtypeStruct((NPAD, 128), F32),
                   jax.ShapeDtypeStruct((1, 1), F32)],
    )(emb4, rinv2, wn, bn, wsd, bsd, natt_pad, nia_pad)


@functools.partial(
    pl.kernel,
    out_type=jax.ShapeDtypeStruct((E, 64), F32),
    mesh=_SC_MESH,
    scratch_types=[
        pltpu.VMEM((128,), jnp.int32), pltpu.VMEM((128,), jnp.int32),
        pltpu.VMEM((128, 128), F32), pltpu.VMEM((128, 128), F32),
        pltpu.VMEM((128, 64), F32),
        pltpu.SemaphoreType.DMA, pltpu.SemaphoreType.DMA,
    ],
)
def _edgehead_kernel(pc_hbm, src_hbm, dst_hbm, out_hbm,
                     sidx_v, didx_v, g1_v, g2_v, ob_v, sem1, sem2):
    cid = lax.axis_index("c")
    sid = lax.axis_index("s")
    w = cid * NS + sid
    nb = jnp.where(w < 2, 40, 39)
    ebase = 128 * jnp.where(w < 2, w * 40, 80 + (w - 2) * 39)

    def batch_body(b, _):
        e0 = ebase + b * 128
        pltpu.sync_copy(src_hbm.at[pl.ds(e0, 128)], sidx_v)
        pltpu.sync_copy(dst_hbm.at[pl.ds(e0, 128)], didx_v)
        cp1 = pltpu.async_copy(pc_hbm.at[sidx_v], g1_v, sem1)
        cp2 = pltpu.async_copy(pc_hbm.at[didx_v], g2_v, sem2)
        cp1.wait()
        cp2.wait()

        def row_body(r, __):
            for j in range(4):
                ob_v[r, pl.ds(j * 16, 16)] = (
                    g1_v[r, pl.ds(j * 16, 16)]
                    + g2_v[r, pl.ds(64 + j * 16, 16)])
            return 0

        lax.fori_loop(0, 128, row_body, 0)
        pltpu.sync_copy(ob_v, out_hbm.at[pl.ds(e0, 128)])
        return 0

    lax.fori_loop(0, nb, batch_body, 0)


_EL_RB = 2000


def _edgeloss_body(oe_ref, rel_ref, ta_ref, relp_ref, tap_ref, loss_ref):
    @pl.when(pl.program_id(0) == 0)
    def _():
        loss_ref[...] = jnp.zeros((1, 1), F32)

    def bce_sum(z, t):
        return jnp.sum(jnp.maximum(z, 0.0) - z * t
                       + jnp.log1p(jnp.exp(-jnp.abs(z))))

    zr = oe_ref[:, 0:19]
    zt = oe_ref[:, 19:52]
    relp_ref[...] = zr
    tap_ref[...] = zt
    loss_ref[...] += (bce_sum(zr, rel_ref[...]) / (E * 19)
                      + bce_sum(zt, ta_ref[...]) / (E * 33))


def _edgeloss_call(oe, erel, eta):
    return pl.pallas_call(
        _edgeloss_body,
        grid=(E // _EL_RB,),
        in_specs=[pl.BlockSpec((_EL_RB, 64), lambda i: (i, 0)),
                  pl.BlockSpec((_EL_RB, 19), lambda i: (i, 0)),
                  pl.BlockSpec((_EL_RB, 33), lambda i: (i, 0))],
        out_specs=[pl.BlockSpec((_EL_RB, 19), lambda i: (i, 0)),
                   pl.BlockSpec((_EL_RB, 33), lambda i: (i, 0)),
                   pl.BlockSpec((1, 1), lambda i: (0, 0))],
        out_shape=[jax.ShapeDtypeStruct((E, 19), F32),
                   jax.ShapeDtypeStruct((E, 33), F32),
                   jax.ShapeDtypeStruct((1, 1), F32)],
    )(oe, erel, eta)


def kernel(x, edge_index, node_att, node_ia, edge_rel, edge_ta,
           W_node1, b_node1, W_node2, b_node2, Wg, bg,
           W_att, b_att, W_ia, b_ia, W_rel, b_rel, W_ta, b_ta):
    src = edge_index[0]
    dst = edge_index[1]

    xw4 = _mlp_call(x, W_node1, b_node1.reshape(1, D), W_node2,
                    b_node2.reshape(1, D), Wg, bg.reshape(1, D))
    deg = _deg_kernel(dst)
    xwn4, rinv2 = _scale_call(xw4, deg.reshape(2, NPAD).T)
    emb2 = _msgpass_kernel(xwn4.reshape(4 * NPAD, 128), src, dst)

    wn = jnp.concatenate([W_att, W_ia, jnp.zeros((D, 115), F32)], axis=1)
    bn = jnp.concatenate([b_att, b_ia, jnp.zeros((115,), F32)]).reshape(1, 128)
    z12 = jnp.zeros((D, 12), F32)
    wsd = jnp.concatenate([W_rel[:D], W_ta[:D], z12,
                           W_rel[D:], W_ta[D:], z12], axis=1)
    bsd = jnp.concatenate([b_rel, b_ta,
                           jnp.zeros((76,), F32)]).reshape(1, 128)
    natt_pad = jnp.concatenate([node_att, jnp.zeros((NPAD - N, 4), F32)],
                               axis=0)
    nia_pad = jnp.concatenate([node_ia, jnp.zeros((NPAD - N, 9), F32)],
                              axis=0)
    pn, pc, loss_n = _heads_call(emb2.reshape(4, NPAD, 128), rinv2,
                                 wn, bn, wsd, bsd, natt_pad, nia_pad)
    oe = _edgehead_kernel(pc, src, dst)
    rel_pred, ta_pred, loss_e = _edgeloss_call(oe, edge_rel, edge_ta)

    loss = loss_n[0, 0] + loss_e[0, 0]
    return (loss, pn[:N, 0:4], pn[:N, 4:13], rel_pred, ta_pred)


# speedup vs baseline: 1.5636x; 1.0260x over previous
import functools

import jax
import jax.numpy as jnp
from jax import lax
from jax.experimental import pallas as pl
from jax.experimental.pallas import tpu as pltpu
from jax.experimental.pallas import tpu_sc as plsc

F32 = jnp.float32
N = 10000
NPAD = 10240
E = 160000
D = 512
NC, NS, L = 2, 16, 16
RB = 1280
NBLK = NPAD // RB

_SC_MESH = plsc.VectorSubcoreMesh(
    core_axis_name="c", subcore_axis_name="s", num_cores=NC, num_subcores=NS)


def _mlp_body(x_ref, w1_ref, b1_ref, w2_ref, b2_ref, wg_ref, bg_ref, out_ref):
    h = jnp.maximum(
        jnp.dot(x_ref[...], w1_ref[...], preferred_element_type=F32)
        + b1_ref[...], 0.0)
    na = jnp.dot(h, w2_ref[...], preferred_element_type=F32) + b2_ref[...]
    xw = jnp.dot(na, wg_ref[...], preferred_element_type=F32) + bg_ref[...]
    for c in range(4):
        out_ref[c] = xw[:, c * 128:(c + 1) * 128]


def _mlp_call(x, w1, b1, w2, b2, wg, bg):
    wspec = pl.BlockSpec((D, D), lambda i: (0, 0))
    bspec = pl.BlockSpec((1, D), lambda i: (0, 0))
    return pl.pallas_call(
        _mlp_body,
        grid=(25,),
        in_specs=[pl.BlockSpec((400, D), lambda i: (i, 0)),
                  wspec, bspec, wspec, bspec, wspec, bspec],
        out_specs=pl.BlockSpec((4, 400, 128), lambda i: (0, i, 0)),
        out_shape=jax.ShapeDtypeStruct((4, NPAD, 128), F32),
    )(x, w1, b1, w2, b2, wg, bg)


_DEG_CB = 1600


@functools.partial(
    pl.kernel,
    out_type=jax.ShapeDtypeStruct((2 * NPAD,), F32),
    mesh=_SC_MESH,
    scratch_types=[
        pltpu.VMEM_SHARED((NPAD,), F32),
        pltpu.VMEM((_DEG_CB,), jnp.int32),
        pltpu.VMEM((_DEG_CB,), F32),
        pltpu.VMEM((320,), F32),
    ],
)
def _deg_kernel(dst_hbm, deg_hbm, hist_sh, didx_v, ones_v, slice_v):
    cid = lax.axis_index("c")
    sid = lax.axis_index("s")
    r0 = sid * 320
    for i in range(20):
        slice_v[pl.ds(i * 16, 16)] = jnp.zeros((16,), F32)
    pltpu.sync_copy(slice_v, hist_sh.at[pl.ds(r0, 320)])
    for i in range(_DEG_CB // 16):
        ones_v[pl.ds(i * 16, 16)] = jnp.full((16,), 1.0, F32)
    plsc.subcore_barrier()

    w = cid * NS + sid
    nb = jnp.where(w < 4, 4, 3)
    ebase = _DEG_CB * jnp.where(w < 4, w * 4, 16 + (w - 4) * 3)

    def chunk_body(k, _):
        pltpu.sync_copy(dst_hbm.at[pl.ds(ebase + k * _DEG_CB, _DEG_CB)],
                        didx_v)
        pltpu.sync_copy(ones_v, hist_sh.at[didx_v], add=True)
        return 0

    lax.fori_loop(0, nb, chunk_body, 0)
    plsc.subcore_barrier()
    pltpu.sync_copy(hist_sh.at[pl.ds(r0, 320)], slice_v)
    pltpu.sync_copy(slice_v, deg_hbm.at[pl.ds(cid * NPAD + r0, 320)])


def _scale_body(xw_ref, deg_ref, xwn_ref, rinv_ref):
    r = lax.rsqrt(deg_ref[:, 0:1] + deg_ref[:, 1:2] + 1.0)
    xwn_ref[0] = xw_ref[0] * r
    rinv_ref[...] = r


def _scale_call(xw4, deg2):
    return pl.pallas_call(
        _scale_body,
        grid=(4, NBLK),
        in_specs=[pl.BlockSpec((1, RB, 128), lambda c, i: (c, i, 0)),
                  pl.BlockSpec((RB, 2), lambda c, i: (i, 0))],
        out_specs=[pl.BlockSpec((1, RB, 128), lambda c, i: (c, i, 0)),
                   pl.BlockSpec((RB, 1), lambda c, i: (i, 0))],
        out_shape=[jax.ShapeDtypeStruct((4, NPAD, 128), F32),
                   jax.ShapeDtypeStruct((NPAD, 1), F32)],
    )(xw4, deg2)


_MP_B = 256


@functools.partial(
    pl.kernel,
    out_type=jax.ShapeDtypeStruct((4 * NPAD, 128), F32),
    mesh=_SC_MESH,
    scratch_types=[
        pltpu.VMEM_SHARED((NPAD, 128), F32),
        pltpu.VMEM((_MP_B, 128), F32),
        pltpu.VMEM((64, 128), F32),
        pltpu.VMEM((_MP_B,), jnp.int32),
        pltpu.VMEM((_MP_B,), jnp.int32),
        pltpu.SemaphoreType.DMA,
    ],
)
def _msgpass_kernel(xwn_hbm, src_hbm, dst_hbm, emb_hbm,
                    s_sh, buf_v, fin_v, sidx_v, didx_v, sem):
    cid = lax.axis_index("c")
    sid = lax.axis_index("s")
    nb = jnp.where(sid < 1, 40, 39)
    ebase = _MP_B * jnp.where(sid < 1, 0, 40 + (sid - 1) * 39)
    r0 = sid * 640
    for p in range(2):
        row_off = (cid * 2 + p) * NPAD
        for k in range(10):
            rr = r0 + k * 64
            pltpu.sync_copy(xwn_hbm.at[pl.ds(row_off + rr, 64)], fin_v)
            pltpu.sync_copy(fin_v, s_sh.at[pl.ds(rr, 64)])
        plsc.subcore_barrier()

        def batch_body(b, _):
            e0 = ebase + b * _MP_B
            pltpu.sync_copy(src_hbm.at[pl.ds(e0, _MP_B)], sidx_v)
            pltpu.sync_copy(dst_hbm.at[pl.ds(e0, _MP_B)], didx_v)
            for i in range(_MP_B // 16):
                sidx_v[pl.ds(i * 16, 16)] = sidx_v[pl.ds(i * 16, 16)] + row_off
            pltpu.async_copy(xwn_hbm.at[sidx_v], buf_v, sem).wait()
            pltpu.sync_copy(buf_v, s_sh.at[didx_v], add=True)
            return 0

        lax.fori_loop(0, nb, batch_body, 0)
        plsc.subcore_barrier()
        for k in range(10):
            rr = r0 + k * 64
            pltpu.sync_copy(s_sh.at[pl.ds(rr, 64)], fin_v)
            pltpu.sync_copy(fin_v, emb_hbm.at[pl.ds(row_off + rr, 64)])
        plsc.subcore_barrier()


def _heads_body(emb_ref, rinv_ref, wn_ref, bn_ref, wsd_ref, bsd_ref,
                natt_ref, nia_ref, pn_ref, pc_ref, loss_ref):
    i = pl.program_id(0)
    c = pl.program_id(1)

    @pl.when(c == 0)
    def _():
        pn_ref[...] = jnp.broadcast_to(bn_ref[...], (RB, 128))
        pc_ref[...] = jnp.broadcast_to(bsd_ref[...], (RB, 128))

    @pl.when((i == 0) & (c == 0))
    def _():
        loss_ref[...] = jnp.zeros((1, 1), F32)

    e = jnp.maximum(emb_ref[0] * rinv_ref[...], 0.0)
    pn_ref[...] += jnp.dot(e, wn_ref[...], preferred_element_type=F32)
    pc_ref[...] += jnp.dot(e, wsd_ref[...], preferred_element_type=F32)

    @pl.when(c == 3)
    def _():
        rows = i * RB + lax.broadcasted_iota(jnp.int32, (RB, 1), 0)
        valid = rows < N

        def bce_sum(z, t):
            v = (jnp.maximum(z, 0.0) - z * t
                 + jnp.log1p(jnp.exp(-jnp.abs(z))))
            return jnp.sum(jnp.where(valid, v, 0.0))

        s_att = bce_sum(pn_ref[:, 0:4], natt_ref[...])
        s_ia = bce_sum(pn_ref[:, 4:13], nia_ref[...])
        loss_ref[...] += s_att / (N * 4) + s_ia / (N * 9)


def _heads_call(emb4, rinv2, wn, bn, wsd, bsd, natt_pad, nia_pad):
    return pl.pallas_call(
        _heads_body,
        grid=(NBLK, 4),
        in_specs=[pl.BlockSpec((1, RB, 128), lambda i, c: (c, i, 0)),
                  pl.BlockSpec((RB, 1), lambda i, c: (i, 0)),
                  pl.BlockSpec((128, 128), lambda i, c: (c, 0)),
                  pl.BlockSpec((1, 128), lambda i, c: (0, 0)),
                  pl.BlockSpec((128, 128), lambda i, c: (c, 0)),
                  pl.BlockSpec((1, 128), lambda i, c: (0, 0)),
                  pl.BlockSpec((RB, 4), lambda i, c: (i, 0)),
                  pl.BlockSpec((RB, 9), lambda i, c: (i, 0))],
        out_specs=[pl.BlockSpec((RB, 128), lambda i, c: (i, 0)),
                   pl.BlockSpec((RB, 128), lambda i, c: (i, 0)),
                   pl.BlockSpec((1, 1), lambda i, c: (0, 0))],
        out_shape=[jax.ShapeDtypeStruct((NPAD, 128), F32),
                   jax.ShapeDtypeStruct((NPAD, 128), F32),
                   jax.ShapeDtypeStruct((1, 1), F32)],
    )(emb4, rinv2, wn, bn, wsd, bsd, natt_pad, nia_pad)


_EH_B = 256


@functools.partial(
    pl.kernel,
    out_type=jax.ShapeDtypeStruct((E, 64), F32),
    mesh=_SC_MESH,
    scratch_types=[
        pltpu.VMEM((_EH_B,), jnp.int32), pltpu.VMEM((_EH_B,), jnp.int32),
        pltpu.VMEM((_EH_B, 128), F32), pltpu.VMEM((_EH_B, 128), F32),
        pltpu.VMEM((_EH_B, 64), F32),
        pltpu.SemaphoreType.DMA, pltpu.SemaphoreType.DMA,
    ],
)
def _edgehead_kernel(pc_hbm, src_hbm, dst_hbm, out_hbm,
                     sidx_v, didx_v, g1_v, g2_v, ob_v, sem1, sem2):
    cid = lax.axis_index("c")
    sid = lax.axis_index("s")
    w = cid * NS + sid
    nb = jnp.where(w < 17, 20, 19)
    ebase = _EH_B * jnp.where(w < 17, w * 20, 340 + (w - 17) * 19)

    def batch_body(b, _):
        e0 = ebase + b * _EH_B
        pltpu.sync_copy(src_hbm.at[pl.ds(e0, _EH_B)], sidx_v)
        pltpu.sync_copy(dst_hbm.at[pl.ds(e0, _EH_B)], didx_v)
        cp1 = pltpu.async_copy(pc_hbm.at[sidx_v], g1_v, sem1)
        cp2 = pltpu.async_copy(pc_hbm.at[didx_v], g2_v, sem2)
        cp1.wait()
        cp2.wait()

        def row_body(r, __):
            for j in range(4):
                ob_v[r, pl.ds(j * 16, 16)] = (
                    g1_v[r, pl.ds(j * 16, 16)]
                    + g2_v[r, pl.ds(64 + j * 16, 16)])
            return 0

        lax.fori_loop(0, _EH_B, row_body, 0)
        pltpu.sync_copy(ob_v, out_hbm.at[pl.ds(e0, _EH_B)])
        return 0

    lax.fori_loop(0, nb, batch_body, 0)


_EL_RB = 2000


def _edgeloss_body(oe_ref, rel_ref, ta_ref, relp_ref, tap_ref, loss_ref):
    @pl.when(pl.program_id(0) == 0)
    def _():
        loss_ref[...] = jnp.zeros((1, 1), F32)

    def bce_sum(z, t):
        return jnp.sum(jnp.maximum(z, 0.0) - z * t
                       + jnp.log1p(jnp.exp(-jnp.abs(z))))

    zr = oe_ref[:, 0:19]
    zt = oe_ref[:, 19:52]
    relp_ref[...] = zr
    tap_ref[...] = zt
    loss_ref[...] += (bce_sum(zr, rel_ref[...]) / (E * 19)
                      + bce_sum(zt, ta_ref[...]) / (E * 33))


def _edgeloss_call(oe, erel, eta):
    return pl.pallas_call(
        _edgeloss_body,
        grid=(E // _EL_RB,),
        in_specs=[pl.BlockSpec((_EL_RB, 64), lambda i: (i, 0)),
                  pl.BlockSpec((_EL_RB, 19), lambda i: (i, 0)),
                  pl.BlockSpec((_EL_RB, 33), lambda i: (i, 0))],
        out_specs=[pl.BlockSpec((_EL_RB, 19), lambda i: (i, 0)),
                   pl.BlockSpec((_EL_RB, 33), lambda i: (i, 0)),
                   pl.BlockSpec((1, 1), lambda i: (0, 0))],
        out_shape=[jax.ShapeDtypeStruct((E, 19), F32),
                   jax.ShapeDtypeStruct((E, 33), F32),
                   jax.ShapeDtypeStruct((1, 1), F32)],
    )(oe, erel, eta)


def kernel(x, edge_index, node_att, node_ia, edge_rel, edge_ta,
           W_node1, b_node1, W_node2, b_node2, Wg, bg,
           W_att, b_att, W_ia, b_ia, W_rel, b_rel, W_ta, b_ta):
    src = edge_index[0]
    dst = edge_index[1]

    xw4 = _mlp_call(x, W_node1, b_node1.reshape(1, D), W_node2,
                    b_node2.reshape(1, D), Wg, bg.reshape(1, D))
    deg = _deg_kernel(dst)
    xwn4, rinv2 = _scale_call(xw4, deg.reshape(2, NPAD).T)
    emb2 = _msgpass_kernel(xwn4.reshape(4 * NPAD, 128), src, dst)

    wn = jnp.concatenate([W_att, W_ia, jnp.zeros((D, 115), F32)], axis=1)
    bn = jnp.concatenate([b_att, b_ia, jnp.zeros((115,), F32)]).reshape(1, 128)
    z12 = jnp.zeros((D, 12), F32)
    wsd = jnp.concatenate([W_rel[:D], W_ta[:D], z12,
                           W_rel[D:], W_ta[D:], z12], axis=1)
    bsd = jnp.concatenate([b_rel, b_ta,
                           jnp.zeros((76,), F32)]).reshape(1, 128)
    natt_pad = jnp.concatenate([node_att, jnp.zeros((NPAD - N, 4), F32)],
                               axis=0)
    nia_pad = jnp.concatenate([node_ia, jnp.zeros((NPAD - N, 9), F32)],
                              axis=0)
    pn, pc, loss_n = _heads_call(emb2.reshape(4, NPAD, 128), rinv2,
                                 wn, bn, wsd, bsd, natt_pad, nia_pad)
    oe = _edgehead_kernel(pc, src, dst)
    rel_pred, ta_pred, loss_e = _edgeloss_call(oe, edge_rel, edge_ta)

    loss = loss_n[0, 0] + loss_e[0, 0]
    return (loss, pn[:N, 0:4], pn[:N, 4:13], rel_pred, ta_pred)
